# baseline (speedup 1.0000x reference)
import functools

import jax
import jax.numpy as jnp
from jax import lax
from jax.experimental import pallas as pl
from jax.experimental.pallas import tpu as pltpu
from jax.experimental.pallas import tpu_sc as plsc

_N = 1024 * 1024
_NC = 2
_NS = 16
_NW = _NC * _NS
_CHUNK = _N // _NW
_L = 16


def _sc_body(w_hbm, n_hbm, o_hbm, w_v, n_v, o_v):
    wid = lax.axis_index("s") * _NC + lax.axis_index("c")
    base = wid * _CHUNK
    pltpu.sync_copy(w_hbm.at[pl.ds(base, _CHUNK)], w_v)
    pltpu.sync_copy(n_hbm.at[pl.ds(base, _CHUNK)], n_v)

    def body(i, carry):
        off = pl.multiple_of(i * _L, _L)
        x = (n_v[pl.ds(off, _L)] - w_v[pl.ds(off, _L)]) * 10.0
        o_v[pl.ds(off, _L)] = 1.0 / (1.0 + jnp.exp(x))
        return carry

    lax.fori_loop(0, _CHUNK // _L, body, 0)
    pltpu.sync_copy(o_v, o_hbm.at[pl.ds(base, _CHUNK)])


_sc_kernel = functools.partial(
    pl.kernel,
    mesh=plsc.VectorSubcoreMesh(core_axis_name="c", subcore_axis_name="s"),
    out_type=jax.ShapeDtypeStruct((_N,), jnp.float32),
    scratch_types=[
        pltpu.VMEM((_CHUNK,), jnp.float32),
        pltpu.VMEM((_CHUNK,), jnp.float32),
        pltpu.VMEM((_CHUNK,), jnp.float32),
    ],
)(_sc_body)


def kernel(weights, noises):
    return _sc_kernel(weights, noises)


# speedup vs baseline: 1.2136x; 1.2136x over previous
import functools

import jax
import jax.numpy as jnp
from jax import lax
from jax.experimental import pallas as pl
from jax.experimental.pallas import tpu as pltpu
from jax.experimental.pallas import tpu_sc as plsc

_N = 1024 * 1024
_NC = 2
_NS = 16
_NW = _NC * _NS
_CHUNK = _N // _NW
_L = 16


def _sc_body(w_hbm, n_hbm, o_hbm, w_v, n_v, o_v):
    wid = lax.axis_index("s") * _NC + lax.axis_index("c")
    base = wid * _CHUNK
    pltpu.sync_copy(w_hbm.at[pl.ds(base, _CHUNK)], w_v)
    pltpu.sync_copy(n_hbm.at[pl.ds(base, _CHUNK)], n_v)

    @plsc.parallel_loop(0, _CHUNK, step=_L, unroll=8)
    def _loop(i):
        x = (n_v[pl.ds(i, _L)] - w_v[pl.ds(i, _L)]) * 10.0
        o_v[pl.ds(i, _L)] = 1.0 / (1.0 + jnp.exp(x))
    pltpu.sync_copy(o_v, o_hbm.at[pl.ds(base, _CHUNK)])


_sc_kernel = functools.partial(
    pl.kernel,
    mesh=plsc.VectorSubcoreMesh(core_axis_name="c", subcore_axis_name="s"),
    out_type=jax.ShapeDtypeStruct((_N,), jnp.float32),
    scratch_types=[
        pltpu.VMEM((_CHUNK,), jnp.float32),
        pltpu.VMEM((_CHUNK,), jnp.float32),
        pltpu.VMEM((_CHUNK,), jnp.float32),
    ],
)(_sc_body)


def kernel(weights, noises):
    return _sc_kernel(weights, noises)
